# baseline (speedup 1.0000x reference)
import functools

import jax
import jax.numpy as jnp
from jax import lax
from jax.experimental import pallas as pl
from jax.experimental.pallas import tpu as pltpu
from jax.experimental.pallas import tpu_sc as plsc

_D = 64
_B = 16384
_NC = 2
_NS = 16
_NW = _NC * _NS
_BPW = _B // _NW
_L = 16
_K = 128

_mesh = plsc.VectorSubcoreMesh(core_axis_name="c", subcore_axis_name="s")


@functools.partial(
    pl.kernel,
    mesh=_mesh,
    out_type=jax.ShapeDtypeStruct((_NW, _D), jnp.float32),
    scratch_types=[
        pltpu.VMEM((_BPW,), jnp.int32),
        pltpu.VMEM((_BPW, _D), jnp.float32),
        pltpu.VMEM((_D,), jnp.float32),
        pltpu.SemaphoreType.DMA,
    ],
)
def _gather_sum(idx_hbm, table_hbm, out_hbm, raw_v, rows_v, acc_v, sem):
    wid = lax.axis_index("s") * _NC + lax.axis_index("c")
    pltpu.sync_copy(idx_hbm.at[pl.ds(wid * _BPW, _BPW)], raw_v)

    def fire_chunk(c):
        v = raw_v[pl.ds(c * _L, _L)]
        for r in range(_L):
            pltpu.make_async_copy(
                table_hbm.at[pl.ds(v[r], 1)],
                rows_v.at[pl.ds(c * _L + r, 1)],
                sem,
            ).start()

    def drain_chunk(c):
        for r in range(_L):
            pltpu.make_async_copy(
                table_hbm.at[pl.ds(0, 1)],
                rows_v.at[pl.ds(c * _L + r, 1)],
                sem,
            ).wait()

    nch = _BPW // _L
    ahead = _K // _L

    def fire_only(c, _):
        fire_chunk(c)
        return 0

    def fire_and_drain(c, _):
        drain_chunk(c - ahead)
        fire_chunk(c)
        return 0

    def drain_only(c, _):
        drain_chunk(c)
        return 0

    lax.fori_loop(0, ahead, fire_only, 0)
    lax.fori_loop(ahead, nch, fire_and_drain, 0)
    lax.fori_loop(nch - ahead, nch, drain_only, 0)

    def row_body(i, carry):
        a0, a1, a2, a3 = carry
        return (
            a0 + rows_v[i, pl.ds(0, _L)],
            a1 + rows_v[i, pl.ds(_L, _L)],
            a2 + rows_v[i, pl.ds(2 * _L, _L)],
            a3 + rows_v[i, pl.ds(3 * _L, _L)],
        )

    z = jnp.zeros((_L,), jnp.float32)
    a0, a1, a2, a3 = lax.fori_loop(0, _BPW, row_body, (z, z, z, z))
    acc_v[pl.ds(0, _L)] = a0
    acc_v[pl.ds(_L, _L)] = a1
    acc_v[pl.ds(2 * _L, _L)] = a2
    acc_v[pl.ds(3 * _L, _L)] = a3
    pltpu.sync_copy(acc_v, out_hbm.at[wid])


def _tail_body(p_ref, w_ref, b_ref, o_ref):
    tot = jnp.sum(p_ref[...], axis=0, keepdims=True)
    h = 1.0 / (1.0 + jnp.exp(-(tot * (1.0 / _B))))
    logits = jnp.sum(w_ref[...] * h, axis=1, keepdims=True) + b_ref[...]
    s = 1.0 / (1.0 + jnp.exp(-logits))
    row = lax.broadcasted_iota(jnp.int32, (8, 1), 0)
    e = jnp.where(row < 2, jnp.exp(s), 0.0)
    o_ref[...] = e / jnp.sum(e)


def kernel(X, emb, W, b):
    partials = _gather_sum(X.astype(jnp.int32), emb)
    h = 1.0 / (1.0 + jnp.exp(-(jnp.sum(partials, axis=0) / _B)))
    s = 1.0 / (1.0 + jnp.exp(-(h @ W.T + b)))
    return jax.nn.softmax(s, axis=0)


# speedup vs baseline: 1.0110x; 1.0110x over previous
import functools

import jax
import jax.numpy as jnp
from jax import lax
from jax.experimental import pallas as pl
from jax.experimental.pallas import tpu as pltpu
from jax.experimental.pallas import tpu_sc as plsc

_D = 64
_B = 16384
_NC = 2
_NS = 16
_NW = _NC * _NS
_BPW = _B // _NW
_L = 16
_K = 128

_mesh = plsc.VectorSubcoreMesh(core_axis_name="c", subcore_axis_name="s")


@functools.partial(
    pl.kernel,
    mesh=_mesh,
    out_type=jax.ShapeDtypeStruct((_NW, _D), jnp.float32),
    scratch_types=[
        pltpu.VMEM((_BPW,), jnp.int32),
        pltpu.VMEM((_BPW, _D), jnp.float32),
        pltpu.VMEM((_D,), jnp.float32),
        pltpu.SemaphoreType.DMA,
    ],
    compiler_params=pltpu.CompilerParams(use_tc_tiling_on_sc=True),
)
def _gather_sum(idx_hbm, table_hbm, out_hbm, raw_v, rows_v, acc_v, sem):
    wid = lax.axis_index("s") * _NC + lax.axis_index("c")
    pltpu.sync_copy(idx_hbm.at[pl.ds(wid * _BPW, _BPW)], raw_v)

    def fire_chunk(c):
        v = raw_v[pl.ds(c * _L, _L)]
        for r in range(_L):
            pltpu.make_async_copy(
                table_hbm.at[pl.ds(v[r], 1)],
                rows_v.at[pl.ds(c * _L + r, 1)],
                sem,
            ).start()

    def drain_chunk(c):
        for r in range(_L):
            pltpu.make_async_copy(
                table_hbm.at[pl.ds(0, 1)],
                rows_v.at[pl.ds(c * _L + r, 1)],
                sem,
            ).wait()

    nch = _BPW // _L
    ahead = _K // _L

    def fire_only(c, _):
        fire_chunk(c)
        return 0

    def fire_and_drain(c, _):
        drain_chunk(c - ahead)
        fire_chunk(c)
        return 0

    def drain_only(c, _):
        drain_chunk(c)
        return 0

    lax.fori_loop(0, ahead, fire_only, 0)
    lax.fori_loop(ahead, nch, fire_and_drain, 0)
    lax.fori_loop(nch - ahead, nch, drain_only, 0)

    def row_body(i, carry):
        a0, a1, a2, a3 = carry
        return (
            a0 + rows_v[i, pl.ds(0, _L)],
            a1 + rows_v[i, pl.ds(_L, _L)],
            a2 + rows_v[i, pl.ds(2 * _L, _L)],
            a3 + rows_v[i, pl.ds(3 * _L, _L)],
        )

    z = jnp.zeros((_L,), jnp.float32)
    a0, a1, a2, a3 = lax.fori_loop(0, _BPW, row_body, (z, z, z, z))
    acc_v[pl.ds(0, _L)] = a0
    acc_v[pl.ds(_L, _L)] = a1
    acc_v[pl.ds(2 * _L, _L)] = a2
    acc_v[pl.ds(3 * _L, _L)] = a3
    pltpu.sync_copy(acc_v, out_hbm.at[wid])


def _tail_body(p_ref, w_ref, b_ref, o_ref):
    tot = jnp.sum(p_ref[...], axis=0, keepdims=True)
    h = 1.0 / (1.0 + jnp.exp(-(tot * (1.0 / _B))))
    logits = jnp.sum(w_ref[...] * h, axis=1, keepdims=True) + b_ref[...]
    s = 1.0 / (1.0 + jnp.exp(-logits))
    row = lax.broadcasted_iota(jnp.int32, (8, 1), 0)
    e = jnp.where(row < 2, jnp.exp(s), 0.0)
    o_ref[...] = e / jnp.sum(e)


def kernel(X, emb, W, b):
    partials = _gather_sum(X.astype(jnp.int32), emb)
    wp = jnp.zeros((8, _D), jnp.float32).at[:2].set(W)
    bp = jnp.zeros((8, 1), jnp.float32).at[:2, 0].set(b)
    out = pl.pallas_call(
        _tail_body,
        out_shape=jax.ShapeDtypeStruct((8, 1), jnp.float32),
    )(partials, wp, bp)
    return out[:2, 0]


# speedup vs baseline: 1.7396x; 1.7207x over previous
import functools

import jax
import jax.numpy as jnp
from jax import lax
from jax.experimental import pallas as pl
from jax.experimental.pallas import tpu as pltpu
from jax.experimental.pallas import tpu_sc as plsc

_D = 64
_B = 16384
_NROW = 1_000_000
_MPAD = 1_000_064
_NC = 2
_NS = 16
_NW = _NC * _NS
_L = 16
_W = 512
_HALF = 499712
_TAILC = 64
_MLOC = 500608
_DBIN = 500480
_ZB = 8192

_mesh = plsc.VectorSubcoreMesh(core_axis_name="c", subcore_axis_name="s")


@functools.partial(
    pl.kernel,
    mesh=_mesh,
    out_type=jax.ShapeDtypeStruct((_NW, _D, _L), jnp.float32),
    scratch_types=[
        pltpu.VMEM_SHARED((_MLOC,), jnp.float32),
        pltpu.VMEM((8, 128), jnp.int32),
        pltpu.VMEM((128,), jnp.float32),
        pltpu.VMEM((_ZB,), jnp.float32),
        pltpu.VMEM((2, _D, _W), jnp.float32),
        pltpu.VMEM((_W,), jnp.float32),
        pltpu.VMEM((_D, _L), jnp.float32),
        pltpu.SemaphoreType.DMA,
        pltpu.SemaphoreType.DMA,
    ],
    compiler_params=pltpu.CompilerParams(use_tc_tiling_on_sc=True),
)
def _count_matvec(idx_hbm, table_hbm, tail_hbm, out_hbm,
                  m_s, idx_v, ones_v, z_v, tbuf, mbuf, part_v, zsem, sem):
    cid = lax.axis_index("c")
    sid = lax.axis_index("s")
    wid = sid * _NC + cid
    base = cid * _HALF
    crange = jnp.where(cid == 0, _HALF, _NROW - _HALF)

    def zfill(i, _):
        z_v[pl.ds(i * _L, _L)] = jnp.zeros((_L,), jnp.float32)
        return 0
    lax.fori_loop(0, _ZB // _L, zfill, 0)
    for j in range(8):
        ones_v[pl.ds(j * _L, _L)] = jnp.ones((_L,), jnp.float32)

    mseg = _MLOC // _NS
    nzc = mseg // _ZB
    zrem = mseg - nzc * _ZB
    for j in range(nzc):
        pltpu.make_async_copy(
            z_v, m_s.at[pl.ds(sid * mseg + j * _ZB, _ZB)], zsem).start()
    pltpu.make_async_copy(
        z_v.at[pl.ds(0, zrem)],
        m_s.at[pl.ds(sid * mseg + nzc * _ZB, zrem)], zsem).start()
    pltpu.sync_copy(idx_hbm.at[sid], idx_v)
    for k in range(8):
        for q in range(8):
            v = idx_v[k, pl.ds(q * _L, _L)]
            loc = v - base
            ok = (loc >= 0) & (loc < crange)
            idx_v[k, pl.ds(q * _L, _L)] = jnp.where(ok, loc, _DBIN)
    for j in range(nzc):
        pltpu.make_async_copy(
            z_v, m_s.at[pl.ds(sid * mseg + j * _ZB, _ZB)], zsem).wait()
    pltpu.make_async_copy(
        z_v.at[pl.ds(0, zrem)],
        m_s.at[pl.ds(sid * mseg + nzc * _ZB, zrem)], zsem).wait()
    plsc.subcore_barrier()

    for k in range(8):
        pltpu.sync_copy(ones_v, m_s.at[idx_v.at[k]], add=True)
    plsc.subcore_barrier()

    def pzero(i, _):
        part_v[i] = jnp.zeros((_L,), jnp.float32)
        return 0
    lax.fori_loop(0, _D, pzero, 0)

    def chunk_col(k):
        return base + (sid + k * _NS) * _W

    def fire(k):
        pltpu.make_async_copy(
            table_hbm.at[:, pl.ds(chunk_col(k), _W)], tbuf.at[jnp.mod(k, 2)],
            sem).start()

    def accumulate(p, loc0):
        pltpu.sync_copy(m_s.at[pl.ds(loc0, _W)], mbuf)
        mv = tuple(mbuf[pl.ds(j * _L, _L)] for j in range(_W // _L))

        def dgroup(dg, _):
            for u in range(4):
                d = dg * 4 + u
                acc = tbuf[p, d, pl.ds(0, _L)] * mv[0]
                for j in range(1, _W // _L):
                    acc = acc + tbuf[p, d, pl.ds(j * _L, _L)] * mv[j]
                plsc.addupdate(part_v.at[d], acc)
            return 0
        lax.fori_loop(0, _D // 4, dgroup, 0)

    kpw = 61
    fire(0)

    def scan_body(k, _):
        pltpu.make_async_copy(
            table_hbm.at[:, pl.ds(0, _W)], tbuf.at[jnp.mod(k, 2)], sem
        ).wait()

        @pl.when(k + 1 < kpw)
        def _():
            fire(k + 1)
        accumulate(jnp.mod(k, 2), chunk_col(k) - base)
        return 0
    lax.fori_loop(0, kpw, scan_body, 0)

    @pl.when((cid == 1) & (sid == 0))
    def _():
        col0 = _HALF + 976 * _W
        pltpu.make_async_copy(
            table_hbm.at[:, pl.ds(col0, _W)], tbuf.at[0], sem).start()
        pltpu.make_async_copy(
            table_hbm.at[:, pl.ds(col0, _W)], tbuf.at[0], sem).wait()
        accumulate(0, col0 - _HALF)

    @pl.when((cid == 1) & (sid == 1))
    def _():
        loc0 = 999936 - _HALF
        pltpu.make_async_copy(
            tail_hbm, tbuf.at[0].at[:, pl.ds(0, 128)], sem).start()
        pltpu.make_async_copy(
            tail_hbm, tbuf.at[0].at[:, pl.ds(0, 128)], sem).wait()
        pltpu.sync_copy(m_s.at[pl.ds(loc0, 128)], mbuf.at[pl.ds(0, 128)])
        mv = tuple(mbuf[pl.ds(j * _L, _L)] for j in range(128 // _L))

        def dtail(dg, _):
            for u in range(4):
                d = dg * 4 + u
                acc = tbuf[0, d, pl.ds(0, _L)] * mv[0]
                for j in range(1, 128 // _L):
                    acc = acc + tbuf[0, d, pl.ds(j * _L, _L)] * mv[j]
                plsc.addupdate(part_v.at[d], acc)
            return 0
        lax.fori_loop(0, _D // 4, dtail, 0)

    pltpu.sync_copy(part_v, out_hbm.at[wid])


def _tail_body(p_ref, w_ref, b_ref, o_ref):
    tot = jnp.sum(p_ref[...], axis=(0, 2))
    h = 1.0 / (1.0 + jnp.exp(-(tot * (1.0 / _B))))
    logits = jnp.sum(w_ref[...] * h[None, :], axis=1, keepdims=True)
    logits = logits + b_ref[...]
    s = 1.0 / (1.0 + jnp.exp(-logits))
    row = lax.broadcasted_iota(jnp.int32, (8, 1), 0)
    e = jnp.where(row < 2, jnp.exp(s), 0.0)
    o_ref[...] = e / jnp.sum(e)


def kernel(X, emb, W, b):
    idx = X.astype(jnp.int32).reshape(_NS, 8, 128)
    tail = jnp.zeros((_D, 128), jnp.float32).at[:, :_TAILC].set(
        emb[999936:].T)
    partials = _count_matvec(idx, emb.T, tail)
    wp = jnp.zeros((8, _D), jnp.float32).at[:2].set(W)
    bp = jnp.zeros((8, 1), jnp.float32).at[:2, 0].set(b)
    out = pl.pallas_call(
        _tail_body,
        out_shape=jax.ShapeDtypeStruct((8, 1), jnp.float32),
    )(partials, wp, bp)
    return out[:2, 0]
